# baseline (speedup 1.0000x reference)
import functools

import jax
import jax.numpy as jnp
from jax import lax
from jax.experimental import pallas as pl
from jax.experimental.pallas import tpu as pltpu
from jax.experimental.pallas import tpu_sc as plsc

N = 10000
E = 320000
EN = E + N
D_IN = 128
D_HID = 128
D_OUT = 64

NC = 2
NS = 16
NW = NC * NS

NP = 10240
NSL = NP // NS
PADN = NP - N

B2 = 128
EPW = 10313
ENP = EPW * NW
KCH = 81
ERW = 88
EPT = ERW * B2
EP = EPT * NW
ER = EP // B2
SC2 = 8
NSUP = 10
NZC = NSL // B2

CB = 1408
K1 = EPT // CB

_mesh = plsc.VectorSubcoreMesh(core_axis_name="c", subcore_axis_name="s")
_f32 = jnp.float32


def _tc_prep_body(x_ref, w_ref, asv_ref, adv_ref, h_ref, a_s_ref, a_d_ref,
                  c_ref):
    n = x_ref.shape[0]
    npad = h_ref.shape[0]
    h = jnp.dot(x_ref[...], w_ref[...], preferred_element_type=_f32)
    h_ref[0:n, :] = h
    h_ref[n:npad, :] = jnp.zeros((npad - n, h.shape[1]), _f32)
    a_s = jnp.dot(h, asv_ref[...], preferred_element_type=_f32)
    a_d = jnp.dot(h, adv_ref[...], preferred_element_type=_f32)
    a_s_ref[0:n, :] = a_s
    a_s_ref[n:npad, :] = jnp.zeros((npad - n, 1), _f32)
    a_d_ref[0:n, :] = a_d
    a_d_ref[n:npad, :] = jnp.zeros((npad - n, 1), _f32)
    c_ref[...] = jnp.full((1, 1), jnp.max(a_s) + jnp.max(a_d), _f32)


def _tc_prep(x, w, asv, adv, d_out):
    return pl.pallas_call(
        _tc_prep_body,
        out_shape=[
            jax.ShapeDtypeStruct((NP, d_out), _f32),
            jax.ShapeDtypeStruct((NP, 1), _f32),
            jax.ShapeDtypeStruct((NP, 1), _f32),
            jax.ShapeDtypeStruct((1, 1), _f32),
        ],
    )(x, w, asv, adv)


def _tc_inv_body(sp_ref, inv_ref):
    sp = sp_ref[...]
    inv_ref[...] = (1.0 / (sp[0] + sp[1] + 1e-16)).reshape(1, NP)


def _tc_inv(sp):
    return pl.pallas_call(
        _tc_inv_body,
        out_shape=jax.ShapeDtypeStruct((1, NP), _f32),
    )(sp)


def _tc_mid_body(op_ref, inv_ref, b_ref, w_ref, asv_ref, adv_ref,
                 x1_ref, h_ref, a_s_ref, a_d_ref, c_ref):
    op = op_ref[...]
    inv = inv_ref[...].reshape(NP, 1)
    x1 = jnp.maximum((op[0] + op[1]) * inv + b_ref[...], 0.0)
    x1_ref[...] = x1
    h = jnp.dot(x1, w_ref[...], preferred_element_type=_f32)
    d = h.shape[1]
    h_ref[0:N, :] = h[0:N, :]
    h_ref[N:NP, :] = jnp.zeros((NP - N, d), _f32)
    a_s = jnp.dot(h, asv_ref[...], preferred_element_type=_f32)
    a_d = jnp.dot(h, adv_ref[...], preferred_element_type=_f32)
    a_s_ref[0:N, :] = a_s[0:N, :]
    a_s_ref[N:NP, :] = jnp.zeros((NP - N, 1), _f32)
    a_d_ref[0:N, :] = a_d[0:N, :]
    a_d_ref[N:NP, :] = jnp.zeros((NP - N, 1), _f32)
    c_ref[...] = jnp.full((1, 1),
                          jnp.max(a_s[0:N, :]) + jnp.max(a_d[0:N, :]), _f32)


def _tc_mid(op, inv, b, w, asv, adv, d_out):
    return pl.pallas_call(
        _tc_mid_body,
        out_shape=[
            jax.ShapeDtypeStruct((NP, D_HID), _f32),
            jax.ShapeDtypeStruct((NP, d_out), _f32),
            jax.ShapeDtypeStruct((NP, 1), _f32),
            jax.ShapeDtypeStruct((NP, 1), _f32),
            jax.ShapeDtypeStruct((1, 1), _f32),
        ],
    )(op, inv, b, w, asv, adv)


def _tc_fin_body(op_ref, inv_ref, b_ref, out_ref):
    op = op_ref[...]
    inv = inv_ref[...].reshape(NP, 1)
    out_ref[...] = (op[0] + op[1]) * inv + b_ref[...]


def _tc_fin(op, inv, b, d_out):
    return pl.pallas_call(
        _tc_fin_body,
        out_shape=jax.ShapeDtypeStruct((NP, d_out), _f32),
    )(op, inv, b)


def _sc_pass1_body(src_hbm, dst_hbm, as_hbm, ad_hbm, c_hbm,
                   ex_hbm, sp_hbm,
                   as_t, ad_t, src_b, dst_b, ex_b, c_b, stage_b, s_sh, sem):
    del sem
    cid = lax.axis_index("c")
    sid = lax.axis_index("s")
    wid = cid * NS + sid

    pltpu.sync_copy(as_hbm, as_t)
    pltpu.sync_copy(ad_hbm, ad_t)
    pltpu.sync_copy(c_hbm, c_b)
    cvec = c_b[...]

    zero16 = jnp.zeros((16,), _f32)

    def zbody(i, carry):
        stage_b[pl.ds(i * 16, 16)] = zero16
        return carry

    lax.fori_loop(0, NSL // 16, zbody, 0)
    pltpu.sync_copy(stage_b, s_sh.at[pl.ds(sid * NSL, NSL)])
    plsc.subcore_barrier()

    ebase = wid * EPT

    def chunk(k, carry):
        base = ebase + k * CB
        pltpu.sync_copy(src_hbm.at[pl.ds(base, CB)], src_b)
        pltpu.sync_copy(dst_hbm.at[pl.ds(base, CB)], dst_b)

        def inner(i, c2):
            off = i * 16
            s16 = src_b[pl.ds(off, 16)]
            d16 = dst_b[pl.ds(off, 16)]
            z = plsc.load_gather(as_t, [s16]) + plsc.load_gather(ad_t, [d16])
            e = jnp.where(z >= 0.0, z, 0.2 * z)
            ex_b[pl.ds(off, 16)] = jnp.exp(e - cvec)
            return c2

        lax.fori_loop(0, CB // 16, inner, 0)
        pltpu.sync_copy(ex_b, ex_hbm.at[pl.ds(base, CB)])
        pltpu.sync_copy(ex_b, s_sh.at[dst_b], add=True)
        return carry

    lax.fori_loop(0, K1, chunk, 0)
    plsc.subcore_barrier()

    nb = sid * NSL
    pltpu.sync_copy(s_sh.at[pl.ds(nb, NSL)], stage_b)
    pltpu.sync_copy(stage_b, sp_hbm.at[pl.ds(cid * NP + nb, NSL)])


_sc_pass1 = functools.partial(
    pl.kernel,
    _sc_pass1_body,
    out_type=[
        jax.ShapeDtypeStruct((EP,), _f32),
        jax.ShapeDtypeStruct((NC * NP,), _f32),
    ],
    mesh=_mesh,
    compiler_params=pltpu.CompilerParams(needs_layout_passes=False),
    scratch_types=[
        pltpu.VMEM((NP,), _f32),
        pltpu.VMEM((NP,), _f32),
        pltpu.VMEM((CB,), jnp.int32),
        pltpu.VMEM((CB,), jnp.int32),
        pltpu.VMEM((CB,), _f32),
        pltpu.VMEM((16,), _f32),
        pltpu.VMEM((NSL,), _f32),
        pltpu.VMEM_SHARED((NP,), _f32),
        pltpu.SemaphoreType.DMA,
    ],
)()


def _make_sc_pass2_body(d):
  dg = d // 16

  def _sc_pass2_body(src_hbm, dst_hbm, ex_hbm, inv_hbm, h_hbm,
                     al_hbm, op_hbm,
                     src_b, dst_b, ex_b, al_b, iv_b, rows_a, rows_b, acc,
                     sem_a, sem_b, sem_s):
    cid = lax.axis_index("c")
    sid = lax.axis_index("s")
    wid = cid * NS + sid

    zero16 = jnp.zeros((16,), _f32)

    def zbody(r, carry):
        for j in range(dg):
            rows_a[r, pl.ds(j * 16, 16)] = zero16
        return carry

    lax.fori_loop(0, B2, zbody, 0)
    nb = sid * NSL
    for z in range(NZC):
        pltpu.sync_copy(rows_a, acc.at[pl.ds(nb + z * B2, B2)])
    plsc.subcore_barrier()

    rbase = wid * ERW
    rows_bufs = (rows_a, rows_b)
    sems = (sem_a, sem_b)

    def fetch_scalars(srow):
        pltpu.sync_copy(src_hbm.at[pl.ds(srow, SC2)], src_b)
        pltpu.sync_copy(dst_hbm.at[pl.ds(srow, SC2)], dst_b)
        pltpu.sync_copy(ex_hbm.at[pl.ds(srow, SC2)], ex_b)

    def alpha_out(srow):
        ivd = [pltpu.async_copy(inv_hbm.at[dst_b.at[c]], iv_b.at[c], sem_s)
               for c in range(SC2)]
        for dsc in ivd:
            dsc.wait()

        def alp_r(r, c2):
            def alp_j(j, c3):
                sl = pl.ds(j * 16, 16)
                al_b[r, sl] = ex_b[r, sl] * iv_b[r, sl]
                return c3
            return lax.fori_loop(0, B2 // 16, alp_j, c2)

        lax.fori_loop(0, SC2, alp_r, 0)
        pltpu.sync_copy(al_b, al_hbm.at[pl.ds(srow, SC2)])

    def scale_chunk(c, cur):
        def scale(i, c2):
            off = i * 16
            ex16 = ex_b[c, pl.ds(off, 16)]
            for l in range(16):
                sc = ex16[l]
                e = off + l
                for j in range(dg):
                    cur[e, pl.ds(j * 16, 16)] = cur[e, pl.ds(j * 16, 16)] * sc
            return c2

        lax.fori_loop(0, B2 // 16, scale, 0)

    def sbody(s, carry):
        srow = rbase + s * SC2
        fetch_scalars(srow)
        descs = {0: pltpu.async_copy(h_hbm.at[src_b.at[0]],
                                     rows_bufs[0], sems[0])}
        alpha_out(srow)
        for c in range(SC2):
            if c + 1 < SC2:
                descs[c + 1] = pltpu.async_copy(
                    h_hbm.at[src_b.at[c + 1]],
                    rows_bufs[(c + 1) % 2], sems[(c + 1) % 2])
            descs[c].wait()
            cur = rows_bufs[c % 2]
            scale_chunk(c, cur)
            pltpu.sync_copy(cur, acc.at[dst_b.at[c]], add=True)
        return carry

    lax.fori_loop(0, NSUP, sbody, 0)

    trow = rbase + NSUP * SC2
    fetch_scalars(trow)
    tdesc = pltpu.async_copy(h_hbm.at[src_b.at[0]], rows_bufs[0], sems[0])
    alpha_out(trow)
    tdesc.wait()
    scale_chunk(0, rows_a)
    pltpu.sync_copy(rows_a, acc.at[dst_b.at[0]], add=True)

    plsc.subcore_barrier()

    for z in range(NZC):
        pltpu.sync_copy(acc.at[pl.ds(nb + z * B2, B2)], rows_a)
        pltpu.sync_copy(rows_a,
                        op_hbm.at[pl.ds(cid * NP + nb + z * B2, B2)])

  return _sc_pass2_body


def _make_sc_pass2(d):
    return pl.kernel(
        _make_sc_pass2_body(d),
        out_type=[
            jax.ShapeDtypeStruct((ER, B2), _f32),
            jax.ShapeDtypeStruct((NC * NP, d), _f32),
        ],
        mesh=_mesh,
        compiler_params=(
            pltpu.CompilerParams(needs_layout_passes=False)
            if d % 128 == 0 else
            pltpu.CompilerParams(needs_layout_passes=False,
                                 use_tc_tiling_on_sc=False)),
        scratch_types=[
            pltpu.VMEM((SC2, B2), jnp.int32),
            pltpu.VMEM((SC2, B2), jnp.int32),
            pltpu.VMEM((SC2, B2), _f32),
            pltpu.VMEM((SC2, B2), _f32),
            pltpu.VMEM((SC2, B2), _f32),
            pltpu.VMEM((B2, d), _f32),
            pltpu.VMEM((B2, d), _f32),
            pltpu.VMEM_SHARED((NP, d), _f32),
            pltpu.SemaphoreType.DMA,
            pltpu.SemaphoreType.DMA,
            pltpu.SemaphoreType.DMA,
        ],
    )


_sc_pass2_h = _make_sc_pass2(D_HID)
_sc_pass2_o = _make_sc_pass2(D_OUT)


def _layer(src1d, dst1d, src2d, dst2d, a_s, a_d, c, h, pass2, d):
    cvec = jnp.broadcast_to(c.reshape(1), (16,))
    ex, sp = _sc_pass1(src1d, dst1d, a_s.reshape(NP), a_d.reshape(NP), cvec)
    inv = _tc_inv(sp.reshape(NC, NP))
    alpha, op = pass2(src2d, dst2d, ex.reshape(ER, B2),
                      inv.reshape(NP), h)
    alpha = alpha.reshape(NW, EPT)[:, :EPW].reshape(ENP)[:EN]
    return alpha, op.reshape(NC, NP, d), inv


def _block_layout(v):
    pad16 = N + (jnp.arange(ENP - EN, dtype=jnp.int32) % PADN)
    w = jnp.concatenate([v, pad16]).reshape(NW, EPW)
    fill = N + (jnp.arange(NW * (EPT - EPW), dtype=jnp.int32)
                % PADN).reshape(NW, EPT - EPW)
    return jnp.concatenate([w, fill], axis=1).reshape(EP)


def kernel(x, edge_index, W1, att_src1, att_dst1, b1,
           W2, att_src2, att_dst2, b2):
    ei = edge_index.astype(jnp.int32)
    loop = jnp.arange(N, dtype=jnp.int32)
    src = _block_layout(jnp.concatenate([ei[0], loop]))
    dst = _block_layout(jnp.concatenate([ei[1], loop]))
    src2d = src.reshape(ER, B2)
    dst2d = dst.reshape(ER, B2)

    h1, as1, ad1, c1 = _tc_prep(x, W1, att_src1.reshape(D_HID, 1),
                                att_dst1.reshape(D_HID, 1), D_HID)
    alpha1, op1, inv1 = _layer(src, dst, src2d, dst2d, as1, ad1, c1, h1,
                               _sc_pass2_h, D_HID)

    emb, h2, as2, ad2, c2 = _tc_mid(op1, inv1, b1.reshape(1, D_HID), W2,
                                    att_src2.reshape(D_OUT, 1),
                                    att_dst2.reshape(D_OUT, 1), D_OUT)
    alpha2, op2, inv2 = _layer(src, dst, src2d, dst2d, as2, ad2, c2, h2,
                               _sc_pass2_o, D_OUT)

    logits = _tc_fin(op2, inv2, b2.reshape(1, D_OUT), D_OUT)
    return (logits[:N], emb[:N], alpha1, alpha2)


# speedup vs baseline: 1.1270x; 1.1270x over previous
import functools

import jax
import jax.numpy as jnp
from jax import lax
from jax.experimental import pallas as pl
from jax.experimental.pallas import tpu as pltpu
from jax.experimental.pallas import tpu_sc as plsc

N = 10000
E = 320000
EN = E + N
D_IN = 128
D_HID = 128
D_OUT = 64

NC = 2
NS = 16
NW = NC * NS

NP = 10240
NSL = NP // NS
PADN = NP - N

B2 = 128
EPW = 10313
ENP = EPW * NW
KCH = 81
ERW = 88
EPT = ERW * B2
EP = EPT * NW
ER = EP // B2
SC2 = 8
NSUP = 10
NZC = NSL // B2

CB = 1408
K1 = EPT // CB

_mesh = plsc.VectorSubcoreMesh(core_axis_name="c", subcore_axis_name="s")
_f32 = jnp.float32


def _tc_prep_body(x_ref, w_ref, asv_ref, adv_ref, h_ref, a_s_ref, a_d_ref,
                  c_ref):
    n = x_ref.shape[0]
    npad = h_ref.shape[0]
    h = jnp.dot(x_ref[...], w_ref[...], preferred_element_type=_f32)
    h_ref[0:n, :] = h
    h_ref[n:npad, :] = jnp.zeros((npad - n, h.shape[1]), _f32)
    a_s = jnp.dot(h, asv_ref[...], preferred_element_type=_f32)
    a_d = jnp.dot(h, adv_ref[...], preferred_element_type=_f32)
    a_s_ref[0:n, :] = a_s
    a_s_ref[n:npad, :] = jnp.zeros((npad - n, 1), _f32)
    a_d_ref[0:n, :] = a_d
    a_d_ref[n:npad, :] = jnp.zeros((npad - n, 1), _f32)
    c_ref[...] = jnp.full((1, 1), jnp.max(a_s) + jnp.max(a_d), _f32)


def _tc_prep(x, w, asv, adv, d_out):
    return pl.pallas_call(
        _tc_prep_body,
        out_shape=[
            jax.ShapeDtypeStruct((NP, d_out), _f32),
            jax.ShapeDtypeStruct((NP, 1), _f32),
            jax.ShapeDtypeStruct((NP, 1), _f32),
            jax.ShapeDtypeStruct((1, 1), _f32),
        ],
    )(x, w, asv, adv)


def _tc_inv_body(sp_ref, inv_ref):
    sp = sp_ref[...]
    inv_ref[...] = (1.0 / (sp[0] + sp[1] + 1e-16)).reshape(1, NP)


def _tc_inv(sp):
    return pl.pallas_call(
        _tc_inv_body,
        out_shape=jax.ShapeDtypeStruct((1, NP), _f32),
    )(sp)


def _tc_mid_body(op_ref, inv_ref, b_ref, w_ref, asv_ref, adv_ref,
                 x1_ref, h_ref, a_s_ref, a_d_ref, c_ref):
    op = op_ref[...]
    inv = inv_ref[...].reshape(NP, 1)
    x1 = jnp.maximum((op[0] + op[1]) * inv + b_ref[...], 0.0)
    x1_ref[...] = x1
    h = jnp.dot(x1, w_ref[...], preferred_element_type=_f32)
    d = h.shape[1]
    h_ref[0:N, :] = h[0:N, :]
    h_ref[N:NP, :] = jnp.zeros((NP - N, d), _f32)
    a_s = jnp.dot(h, asv_ref[...], preferred_element_type=_f32)
    a_d = jnp.dot(h, adv_ref[...], preferred_element_type=_f32)
    a_s_ref[0:N, :] = a_s[0:N, :]
    a_s_ref[N:NP, :] = jnp.zeros((NP - N, 1), _f32)
    a_d_ref[0:N, :] = a_d[0:N, :]
    a_d_ref[N:NP, :] = jnp.zeros((NP - N, 1), _f32)
    c_ref[...] = jnp.full((1, 1),
                          jnp.max(a_s[0:N, :]) + jnp.max(a_d[0:N, :]), _f32)


def _tc_mid(op, inv, b, w, asv, adv, d_out):
    return pl.pallas_call(
        _tc_mid_body,
        out_shape=[
            jax.ShapeDtypeStruct((NP, D_HID), _f32),
            jax.ShapeDtypeStruct((NP, d_out), _f32),
            jax.ShapeDtypeStruct((NP, 1), _f32),
            jax.ShapeDtypeStruct((NP, 1), _f32),
            jax.ShapeDtypeStruct((1, 1), _f32),
        ],
    )(op, inv, b, w, asv, adv)


def _tc_fin_body(op_ref, inv_ref, b_ref, out_ref):
    op = op_ref[...]
    inv = inv_ref[...].reshape(NP, 1)
    out_ref[...] = (op[0] + op[1]) * inv + b_ref[...]


def _tc_fin(op, inv, b, d_out):
    return pl.pallas_call(
        _tc_fin_body,
        out_shape=jax.ShapeDtypeStruct((NP, d_out), _f32),
    )(op, inv, b)


def _sc_pass1_body(src_hbm, dst_hbm, as_hbm, ad_hbm, c_hbm,
                   ex_hbm, sp_hbm,
                   as_t, ad_t, src_b, dst_b, ex_b, c_b, stage_b, s_sh, sem):
    del sem
    cid = lax.axis_index("c")
    sid = lax.axis_index("s")
    wid = cid * NS + sid

    pltpu.sync_copy(as_hbm, as_t)
    pltpu.sync_copy(ad_hbm, ad_t)
    pltpu.sync_copy(c_hbm, c_b)
    cvec = c_b[...]

    zero16 = jnp.zeros((16,), _f32)

    def zbody(i, carry):
        stage_b[pl.ds(i * 16, 16)] = zero16
        return carry

    lax.fori_loop(0, NSL // 16, zbody, 0)
    pltpu.sync_copy(stage_b, s_sh.at[pl.ds(sid * NSL, NSL)])
    plsc.subcore_barrier()

    ebase = wid * EPT

    def chunk(k, carry):
        base = ebase + k * CB
        pltpu.sync_copy(src_hbm.at[pl.ds(base, CB)], src_b)
        pltpu.sync_copy(dst_hbm.at[pl.ds(base, CB)], dst_b)

        def inner(i, c2):
            off = i * 16
            s16 = src_b[pl.ds(off, 16)]
            d16 = dst_b[pl.ds(off, 16)]
            z = plsc.load_gather(as_t, [s16]) + plsc.load_gather(ad_t, [d16])
            e = jnp.where(z >= 0.0, z, 0.2 * z)
            ex_b[pl.ds(off, 16)] = jnp.exp(e - cvec)
            return c2

        lax.fori_loop(0, CB // 16, inner, 0)
        pltpu.sync_copy(ex_b, ex_hbm.at[pl.ds(base, CB)])
        pltpu.sync_copy(ex_b, s_sh.at[dst_b], add=True)
        return carry

    lax.fori_loop(0, K1, chunk, 0)
    plsc.subcore_barrier()

    nb = sid * NSL
    pltpu.sync_copy(s_sh.at[pl.ds(nb, NSL)], stage_b)
    pltpu.sync_copy(stage_b, sp_hbm.at[pl.ds(cid * NP + nb, NSL)])


_sc_pass1 = functools.partial(
    pl.kernel,
    _sc_pass1_body,
    out_type=[
        jax.ShapeDtypeStruct((EP,), _f32),
        jax.ShapeDtypeStruct((NC * NP,), _f32),
    ],
    mesh=_mesh,
    compiler_params=pltpu.CompilerParams(needs_layout_passes=False),
    scratch_types=[
        pltpu.VMEM((NP,), _f32),
        pltpu.VMEM((NP,), _f32),
        pltpu.VMEM((CB,), jnp.int32),
        pltpu.VMEM((CB,), jnp.int32),
        pltpu.VMEM((CB,), _f32),
        pltpu.VMEM((16,), _f32),
        pltpu.VMEM((NSL,), _f32),
        pltpu.VMEM_SHARED((NP,), _f32),
        pltpu.SemaphoreType.DMA,
    ],
)()


def _make_sc_pass2_body(d):
  dg = d // 16

  def _sc_pass2_body(src_hbm, dst_hbm, ex_hbm, inv_hbm, h_hbm,
                     al_hbm, op_hbm,
                     src_b, dst_b, ex_b, al_b, iv_b, rows_a, rows_b, acc,
                     sem_a, sem_b, sem_s, sem_c):
    cid = lax.axis_index("c")
    sid = lax.axis_index("s")
    wid = cid * NS + sid

    zero16 = jnp.zeros((16,), _f32)

    def zbody(r, carry):
        for j in range(dg):
            rows_a[r, pl.ds(j * 16, 16)] = zero16
        return carry

    lax.fori_loop(0, B2, zbody, 0)
    nb = sid * NSL
    for z in range(NZC):
        pltpu.sync_copy(rows_a, acc.at[pl.ds(nb + z * B2, B2)])
    plsc.subcore_barrier()

    rbase = wid * ERW
    rows_bufs = (rows_a, rows_b)
    sems = (sem_a, sem_b)

    def fetch_scalars(srow):
        pltpu.sync_copy(src_hbm.at[pl.ds(srow, SC2)], src_b)
        pltpu.sync_copy(dst_hbm.at[pl.ds(srow, SC2)], dst_b)
        pltpu.sync_copy(ex_hbm.at[pl.ds(srow, SC2)], ex_b)

    def alpha_out(srow):
        ivd = [pltpu.async_copy(inv_hbm.at[dst_b.at[c]], iv_b.at[c], sem_s)
               for c in range(SC2)]
        for dsc in ivd:
            dsc.wait()

        def alp_r(r, c2):
            def alp_j(j, c3):
                sl = pl.ds(j * 16, 16)
                al_b[r, sl] = ex_b[r, sl] * iv_b[r, sl]
                return c3
            return lax.fori_loop(0, B2 // 16, alp_j, c2)

        lax.fori_loop(0, SC2, alp_r, 0)
        pltpu.sync_copy(al_b, al_hbm.at[pl.ds(srow, SC2)])

    def scale_chunk(c, cur):
        def scale(i, c2):
            off = i * 16
            ex16 = ex_b[c, pl.ds(off, 16)]
            for l in range(16):
                sc = ex16[l]
                e = off + l
                for j in range(dg):
                    cur[e, pl.ds(j * 16, 16)] = cur[e, pl.ds(j * 16, 16)] * sc
            return c2

        lax.fori_loop(0, B2 // 16, scale, 0)

    def sbody(s, carry):
        srow = rbase + s * SC2
        fetch_scalars(srow)
        descs = {0: pltpu.async_copy(h_hbm.at[src_b.at[0]],
                                     rows_bufs[0], sems[0])}
        alpha_out(srow)
        sdescs = {}
        for c in range(SC2):
            if c + 1 < SC2:
                if c >= 1:
                    sdescs[c - 1].wait()
                descs[c + 1] = pltpu.async_copy(
                    h_hbm.at[src_b.at[c + 1]],
                    rows_bufs[(c + 1) % 2], sems[(c + 1) % 2])
            descs[c].wait()
            cur = rows_bufs[c % 2]
            scale_chunk(c, cur)
            sdescs[c] = pltpu.async_copy(cur, acc.at[dst_b.at[c]], sem_c,
                                         add=True)
        sdescs[SC2 - 2].wait()
        sdescs[SC2 - 1].wait()
        return carry

    lax.fori_loop(0, NSUP, sbody, 0)

    trow = rbase + NSUP * SC2
    fetch_scalars(trow)
    tdesc = pltpu.async_copy(h_hbm.at[src_b.at[0]], rows_bufs[0], sems[0])
    alpha_out(trow)
    tdesc.wait()
    scale_chunk(0, rows_a)
    pltpu.sync_copy(rows_a, acc.at[dst_b.at[0]], add=True)

    plsc.subcore_barrier()

    for z in range(NZC):
        pltpu.sync_copy(acc.at[pl.ds(nb + z * B2, B2)], rows_a)
        pltpu.sync_copy(rows_a,
                        op_hbm.at[pl.ds(cid * NP + nb + z * B2, B2)])

  return _sc_pass2_body


def _make_sc_pass2(d):
    return pl.kernel(
        _make_sc_pass2_body(d),
        out_type=[
            jax.ShapeDtypeStruct((ER, B2), _f32),
            jax.ShapeDtypeStruct((NC * NP, d), _f32),
        ],
        mesh=_mesh,
        compiler_params=(
            pltpu.CompilerParams(needs_layout_passes=False)
            if d % 128 == 0 else
            pltpu.CompilerParams(needs_layout_passes=False,
                                 use_tc_tiling_on_sc=False)),
        scratch_types=[
            pltpu.VMEM((SC2, B2), jnp.int32),
            pltpu.VMEM((SC2, B2), jnp.int32),
            pltpu.VMEM((SC2, B2), _f32),
            pltpu.VMEM((SC2, B2), _f32),
            pltpu.VMEM((SC2, B2), _f32),
            pltpu.VMEM((B2, d), _f32),
            pltpu.VMEM((B2, d), _f32),
            pltpu.VMEM_SHARED((NP, d), _f32),
            pltpu.SemaphoreType.DMA,
            pltpu.SemaphoreType.DMA,
            pltpu.SemaphoreType.DMA,
            pltpu.SemaphoreType.DMA,
        ],
    )


_sc_pass2_h = _make_sc_pass2(D_HID)


def _layer(src1d, dst1d, src2d, dst2d, a_s, a_d, c, h, pass2, d):
    cvec = jnp.broadcast_to(c.reshape(1), (16,))
    ex, sp = _sc_pass1(src1d, dst1d, a_s.reshape(NP), a_d.reshape(NP), cvec)
    inv = _tc_inv(sp.reshape(NC, NP))
    alpha, op = pass2(src2d, dst2d, ex.reshape(ER, B2),
                      inv.reshape(NP), h)
    alpha = alpha.reshape(NW, EPT)[:, :EPW].reshape(ENP)[:EN]
    return alpha, op.reshape(NC, NP, d), inv


def _block_layout(v):
    pad16 = N + (jnp.arange(ENP - EN, dtype=jnp.int32) % PADN)
    w = jnp.concatenate([v, pad16]).reshape(NW, EPW)
    fill = N + (jnp.arange(NW * (EPT - EPW), dtype=jnp.int32)
                % PADN).reshape(NW, EPT - EPW)
    return jnp.concatenate([w, fill], axis=1).reshape(EP)


def kernel(x, edge_index, W1, att_src1, att_dst1, b1,
           W2, att_src2, att_dst2, b2):
    ei = edge_index.astype(jnp.int32)
    loop = jnp.arange(N, dtype=jnp.int32)
    src = _block_layout(jnp.concatenate([ei[0], loop]))
    dst = _block_layout(jnp.concatenate([ei[1], loop]))
    src2d = src.reshape(ER, B2)
    dst2d = dst.reshape(ER, B2)

    w2p = jnp.pad(W2, ((0, 0), (0, D_HID - D_OUT)))
    as2p = jnp.pad(att_src2, (0, D_HID - D_OUT)).reshape(D_HID, 1)
    ad2p = jnp.pad(att_dst2, (0, D_HID - D_OUT)).reshape(D_HID, 1)

    h1, as1, ad1, c1 = _tc_prep(x, W1, att_src1.reshape(D_HID, 1),
                                att_dst1.reshape(D_HID, 1), D_HID)
    alpha1, op1, inv1 = _layer(src, dst, src2d, dst2d, as1, ad1, c1, h1,
                               _sc_pass2_h, D_HID)

    emb, h2, as2, ad2, c2 = _tc_mid(op1, inv1, b1.reshape(1, D_HID), w2p,
                                    as2p, ad2p, D_HID)
    alpha2, op2, inv2 = _layer(src, dst, src2d, dst2d, as2, ad2, c2, h2,
                               _sc_pass2_h, D_HID)

    logits = _tc_fin(op2[:, :, :D_OUT], inv2, b2.reshape(1, D_OUT), D_OUT)
    return (logits[:N], emb[:N], alpha1, alpha2)


# speedup vs baseline: 1.4473x; 1.2842x over previous
import functools

import jax
import jax.numpy as jnp
from jax import lax
from jax.experimental import pallas as pl
from jax.experimental.pallas import tpu as pltpu
from jax.experimental.pallas import tpu_sc as plsc

N = 10000
E = 320000
EN = E + N
D_IN = 128
D_HID = 128
D_OUT = 64

NC = 2
NS = 16
NW = NC * NS

NP = 10240
NSL = NP // NS
PADN = NP - N

B2 = 128
EPW = 10313
ENP = EPW * NW
KCH = 81
ERW = 88
EPT = ERW * B2
EP = EPT * NW
ER = EP // B2
SC2 = 8
NSUP = 10
NZC = NSL // B2

CB = 1408
K1 = EPT // CB

_mesh = plsc.VectorSubcoreMesh(core_axis_name="c", subcore_axis_name="s")
_f32 = jnp.float32


def _tc_prep_body(x_ref, w_ref, asv_ref, adv_ref, h_ref, a_s_ref, a_d_ref,
                  c_ref):
    n = x_ref.shape[0]
    npad = h_ref.shape[0]
    h = jnp.dot(x_ref[...], w_ref[...], preferred_element_type=_f32)
    h_ref[0:n, :] = h
    h_ref[n:npad, :] = jnp.zeros((npad - n, h.shape[1]), _f32)
    a_s = jnp.dot(h, asv_ref[...], preferred_element_type=_f32)
    a_d = jnp.dot(h, adv_ref[...], preferred_element_type=_f32)
    a_s_ref[0:n, :] = a_s
    a_s_ref[n:npad, :] = jnp.zeros((npad - n, 1), _f32)
    a_d_ref[0:n, :] = a_d
    a_d_ref[n:npad, :] = jnp.zeros((npad - n, 1), _f32)
    c_ref[...] = jnp.full((1, 1), jnp.max(a_s) + jnp.max(a_d), _f32)


def _tc_prep(x, w, asv, adv, d_out):
    return pl.pallas_call(
        _tc_prep_body,
        out_shape=[
            jax.ShapeDtypeStruct((NP, d_out), _f32),
            jax.ShapeDtypeStruct((NP, 1), _f32),
            jax.ShapeDtypeStruct((NP, 1), _f32),
            jax.ShapeDtypeStruct((1, 1), _f32),
        ],
    )(x, w, asv, adv)


def _tc_inv_body(sp_ref, inv_ref):
    sp = sp_ref[...]
    inv_ref[...] = (1.0 / (sp[0] + sp[1] + 1e-16)).reshape(1, NP)


def _tc_inv(sp):
    return pl.pallas_call(
        _tc_inv_body,
        out_shape=jax.ShapeDtypeStruct((1, NP), _f32),
    )(sp)


def _tc_mid_body(op_ref, inv_ref, b_ref, w_ref, asv_ref, adv_ref,
                 x1_ref, h_ref, a_s_ref, a_d_ref, c_ref):
    op = op_ref[...]
    inv = inv_ref[...].reshape(NP, 1)
    x1 = jnp.maximum((op[0] + op[1]) * inv + b_ref[...], 0.0)
    x1_ref[...] = x1
    h = jnp.dot(x1, w_ref[...], preferred_element_type=_f32)
    d = h.shape[1]
    h_ref[0:N, :] = h[0:N, :]
    h_ref[N:NP, :] = jnp.zeros((NP - N, d), _f32)
    a_s = jnp.dot(h, asv_ref[...], preferred_element_type=_f32)
    a_d = jnp.dot(h, adv_ref[...], preferred_element_type=_f32)
    a_s_ref[0:N, :] = a_s[0:N, :]
    a_s_ref[N:NP, :] = jnp.zeros((NP - N, 1), _f32)
    a_d_ref[0:N, :] = a_d[0:N, :]
    a_d_ref[N:NP, :] = jnp.zeros((NP - N, 1), _f32)
    c_ref[...] = jnp.full((1, 1),
                          jnp.max(a_s[0:N, :]) + jnp.max(a_d[0:N, :]), _f32)


def _tc_mid(op, inv, b, w, asv, adv, d_out):
    return pl.pallas_call(
        _tc_mid_body,
        out_shape=[
            jax.ShapeDtypeStruct((NP, D_HID), _f32),
            jax.ShapeDtypeStruct((NP, d_out), _f32),
            jax.ShapeDtypeStruct((NP, 1), _f32),
            jax.ShapeDtypeStruct((NP, 1), _f32),
            jax.ShapeDtypeStruct((1, 1), _f32),
        ],
    )(op, inv, b, w, asv, adv)


def _tc_fin_body(op_ref, inv_ref, b_ref, out_ref):
    op = op_ref[...]
    inv = inv_ref[...].reshape(NP, 1)
    out_ref[...] = (op[0] + op[1]) * inv + b_ref[...]


def _tc_fin(op, inv, b, d_out):
    return pl.pallas_call(
        _tc_fin_body,
        out_shape=jax.ShapeDtypeStruct((NP, d_out), _f32),
    )(op, inv, b)


def _sc_pass1_body(src_hbm, dst_hbm, as_hbm, ad_hbm, c_hbm,
                   ex_hbm, sp_hbm,
                   as_t, ad_t, src_b, dst_b, ex_b, c_b, stage_b, s_sh, sem):
    del sem
    cid = lax.axis_index("c")
    sid = lax.axis_index("s")
    wid = cid * NS + sid

    pltpu.sync_copy(as_hbm, as_t)
    pltpu.sync_copy(ad_hbm, ad_t)
    pltpu.sync_copy(c_hbm, c_b)
    cvec = c_b[...]

    zero16 = jnp.zeros((16,), _f32)

    def zbody(i, carry):
        stage_b[pl.ds(i * 16, 16)] = zero16
        return carry

    lax.fori_loop(0, NSL // 16, zbody, 0)
    pltpu.sync_copy(stage_b, s_sh.at[pl.ds(sid * NSL, NSL)])
    plsc.subcore_barrier()

    ebase = wid * EPT

    def chunk(k, carry):
        base = ebase + k * CB
        pltpu.sync_copy(src_hbm.at[pl.ds(base, CB)], src_b)
        pltpu.sync_copy(dst_hbm.at[pl.ds(base, CB)], dst_b)

        def inner(i, c2):
            off = i * 16
            s16 = src_b[pl.ds(off, 16)]
            d16 = dst_b[pl.ds(off, 16)]
            z = plsc.load_gather(as_t, [s16]) + plsc.load_gather(ad_t, [d16])
            e = jnp.where(z >= 0.0, z, 0.2 * z)
            ex_b[pl.ds(off, 16)] = jnp.exp(e - cvec)
            return c2

        lax.fori_loop(0, CB // 16, inner, 0)
        pltpu.sync_copy(ex_b, ex_hbm.at[pl.ds(base, CB)])
        pltpu.sync_copy(ex_b, s_sh.at[dst_b], add=True)
        return carry

    lax.fori_loop(0, K1, chunk, 0)
    plsc.subcore_barrier()

    nb = sid * NSL
    pltpu.sync_copy(s_sh.at[pl.ds(nb, NSL)], stage_b)
    pltpu.sync_copy(stage_b, sp_hbm.at[pl.ds(cid * NP + nb, NSL)])


_sc_pass1 = functools.partial(
    pl.kernel,
    _sc_pass1_body,
    out_type=[
        jax.ShapeDtypeStruct((EP,), _f32),
        jax.ShapeDtypeStruct((NC * NP,), _f32),
    ],
    mesh=_mesh,
    compiler_params=pltpu.CompilerParams(needs_layout_passes=False),
    scratch_types=[
        pltpu.VMEM((NP,), _f32),
        pltpu.VMEM((NP,), _f32),
        pltpu.VMEM((CB,), jnp.int32),
        pltpu.VMEM((CB,), jnp.int32),
        pltpu.VMEM((CB,), _f32),
        pltpu.VMEM((16,), _f32),
        pltpu.VMEM((NSL,), _f32),
        pltpu.VMEM_SHARED((NP,), _f32),
        pltpu.SemaphoreType.DMA,
    ],
)()


def _make_sc_pass2_body(d):
  dg = d // 16

  def _sc_pass2_body(src_hbm, dst_hbm, ex_hbm, inv_hbm, h_hbm,
                     al_hbm, op_hbm,
                     src_b, dst_b, ex_b, al_b, iv_b, rows_a, rows_b, acc,
                     inv_sh, sem_a, sem_b, sem_s, sem_c):
    cid = lax.axis_index("c")
    sid = lax.axis_index("s")
    wid = cid * NS + sid

    zero16 = jnp.zeros((16,), _f32)

    def zbody(r, carry):
        for j in range(dg):
            rows_a[r, pl.ds(j * 16, 16)] = zero16
        return carry

    lax.fori_loop(0, B2, zbody, 0)
    nb = sid * NSL
    for z in range(NZC):
        pltpu.sync_copy(rows_a, acc.at[pl.ds(nb + z * B2, B2)])
    pltpu.sync_copy(inv_hbm.at[pl.ds(nb, NSL)], inv_sh.at[pl.ds(nb, NSL)])
    plsc.subcore_barrier()

    rbase = wid * ERW
    rows_bufs = (rows_a, rows_b)
    sems = (sem_a, sem_b)

    def fetch_scalars(srow):
        pltpu.sync_copy(src_hbm.at[pl.ds(srow, SC2)], src_b)
        pltpu.sync_copy(dst_hbm.at[pl.ds(srow, SC2)], dst_b)
        pltpu.sync_copy(ex_hbm.at[pl.ds(srow, SC2)], ex_b)

    def alpha_out(srow):
        ivd = [pltpu.async_copy(inv_sh.at[dst_b.at[c]], iv_b.at[c], sem_s)
               for c in range(SC2)]
        for dsc in ivd:
            dsc.wait()

        def alp_r(r, c2):
            def alp_j(j, c3):
                sl = pl.ds(j * 16, 16)
                al_b[r, sl] = ex_b[r, sl] * iv_b[r, sl]
                return c3
            return lax.fori_loop(0, B2 // 16, alp_j, c2)

        lax.fori_loop(0, SC2, alp_r, 0)
        pltpu.sync_copy(al_b, al_hbm.at[pl.ds(srow, SC2)])

    def scale_chunk(c, cur):
        def scale(i, c2):
            off = i * 16
            ex16 = ex_b[c, pl.ds(off, 16)]
            for l in range(16):
                sc = ex16[l]
                e = off + l
                for j in range(dg):
                    cur[e, pl.ds(j * 16, 16)] = cur[e, pl.ds(j * 16, 16)] * sc
            return c2

        lax.fori_loop(0, B2 // 16, scale, 0)

    def sbody(s, carry):
        srow = rbase + s * SC2
        fetch_scalars(srow)
        descs = {0: pltpu.async_copy(h_hbm.at[src_b.at[0]],
                                     rows_bufs[0], sems[0])}
        alpha_out(srow)
        sdescs = {}
        for c in range(SC2):
            if c + 1 < SC2:
                if c >= 1:
                    sdescs[c - 1].wait()
                descs[c + 1] = pltpu.async_copy(
                    h_hbm.at[src_b.at[c + 1]],
                    rows_bufs[(c + 1) % 2], sems[(c + 1) % 2])
            descs[c].wait()
            cur = rows_bufs[c % 2]
            scale_chunk(c, cur)
            sdescs[c] = pltpu.async_copy(cur, acc.at[dst_b.at[c]], sem_c,
                                         add=True)
        sdescs[SC2 - 2].wait()
        sdescs[SC2 - 1].wait()
        return carry

    lax.fori_loop(0, NSUP, sbody, 0)

    trow = rbase + NSUP * SC2
    fetch_scalars(trow)
    tdesc = pltpu.async_copy(h_hbm.at[src_b.at[0]], rows_bufs[0], sems[0])
    alpha_out(trow)
    tdesc.wait()
    scale_chunk(0, rows_a)
    pltpu.sync_copy(rows_a, acc.at[dst_b.at[0]], add=True)

    plsc.subcore_barrier()

    for z in range(NZC):
        pltpu.sync_copy(acc.at[pl.ds(nb + z * B2, B2)], rows_a)
        pltpu.sync_copy(rows_a,
                        op_hbm.at[pl.ds(cid * NP + nb + z * B2, B2)])

  return _sc_pass2_body


def _make_sc_pass2(d):
    return pl.kernel(
        _make_sc_pass2_body(d),
        out_type=[
            jax.ShapeDtypeStruct((ER, B2), _f32),
            jax.ShapeDtypeStruct((NC * NP, d), _f32),
        ],
        mesh=_mesh,
        compiler_params=(
            pltpu.CompilerParams(needs_layout_passes=False)
            if d % 128 == 0 else
            pltpu.CompilerParams(needs_layout_passes=False,
                                 use_tc_tiling_on_sc=False)),
        scratch_types=[
            pltpu.VMEM((SC2, B2), jnp.int32),
            pltpu.VMEM((SC2, B2), jnp.int32),
            pltpu.VMEM((SC2, B2), _f32),
            pltpu.VMEM((SC2, B2), _f32),
            pltpu.VMEM((SC2, B2), _f32),
            pltpu.VMEM((B2, d), _f32),
            pltpu.VMEM((B2, d), _f32),
            pltpu.VMEM_SHARED((NP, d), _f32),
            pltpu.VMEM_SHARED((NP,), _f32),
            pltpu.SemaphoreType.DMA,
            pltpu.SemaphoreType.DMA,
            pltpu.SemaphoreType.DMA,
            pltpu.SemaphoreType.DMA,
        ],
    )


_sc_pass2_h = _make_sc_pass2(D_HID)


def _layer(src1d, dst1d, src2d, dst2d, a_s, a_d, c, h, pass2, d):
    cvec = jnp.broadcast_to(c.reshape(1), (16,))
    ex, sp = _sc_pass1(src1d, dst1d, a_s.reshape(NP), a_d.reshape(NP), cvec)
    inv = _tc_inv(sp.reshape(NC, NP))
    alpha, op = pass2(src2d, dst2d, ex.reshape(ER, B2),
                      inv.reshape(NP), h)
    alpha = alpha.reshape(NW, EPT)[:, :EPW].reshape(ENP)[:EN]
    return alpha, op.reshape(NC, NP, d), inv


def _block_layout(v):
    pad16 = N + (jnp.arange(ENP - EN, dtype=jnp.int32) % PADN)
    w = jnp.concatenate([v, pad16]).reshape(NW, EPW)
    fill = N + (jnp.arange(NW * (EPT - EPW), dtype=jnp.int32)
                % PADN).reshape(NW, EPT - EPW)
    return jnp.concatenate([w, fill], axis=1).reshape(EP)


def kernel(x, edge_index, W1, att_src1, att_dst1, b1,
           W2, att_src2, att_dst2, b2):
    ei = edge_index.astype(jnp.int32)
    loop = jnp.arange(N, dtype=jnp.int32)
    src = _block_layout(jnp.concatenate([ei[0], loop]))
    dst = _block_layout(jnp.concatenate([ei[1], loop]))
    src2d = src.reshape(ER, B2)
    dst2d = dst.reshape(ER, B2)

    w2p = jnp.pad(W2, ((0, 0), (0, D_HID - D_OUT)))
    as2p = jnp.pad(att_src2, (0, D_HID - D_OUT)).reshape(D_HID, 1)
    ad2p = jnp.pad(att_dst2, (0, D_HID - D_OUT)).reshape(D_HID, 1)

    h1, as1, ad1, c1 = _tc_prep(x, W1, att_src1.reshape(D_HID, 1),
                                att_dst1.reshape(D_HID, 1), D_HID)
    alpha1, op1, inv1 = _layer(src, dst, src2d, dst2d, as1, ad1, c1, h1,
                               _sc_pass2_h, D_HID)

    emb, h2, as2, ad2, c2 = _tc_mid(op1, inv1, b1.reshape(1, D_HID), w2p,
                                    as2p, ad2p, D_HID)
    alpha2, op2, inv2 = _layer(src, dst, src2d, dst2d, as2, ad2, c2, h2,
                               _sc_pass2_h, D_HID)

    logits = _tc_fin(op2[:, :, :D_OUT], inv2, b2.reshape(1, D_OUT), D_OUT)
    return (logits[:N], emb[:N], alpha1, alpha2)
